# baseline (speedup 1.0000x reference)
import functools

import jax
import jax.numpy as jnp
from jax import lax
from jax.experimental import pallas as pl
from jax.experimental.pallas import tpu as pltpu
from jax.experimental.pallas import tpu_sc as plsc

N = 10000
D = 128
E = 320000
NC, NS = 2, 16
NW = NC * NS
CH = 128
CPT = 80
EPAD = NW * CPT * CH
NPAD = 10240
NPT = NPAD // NS
BLK = 640
GRID = NPAD // BLK

_MESH = plsc.VectorSubcoreMesh(
    core_axis_name="c", subcore_axis_name="s", num_cores=NC, num_subcores=NS)


def _deg_body(ei_hbm, deg_out, row_v, ones_v, zv, deg_s):
    c = lax.axis_index("c")
    s = lax.axis_index("s")
    w = c * NS + s

    def zb(r, carry):
        zv[pl.ds(r * 16, 16)] = jnp.zeros((16,), jnp.float32)
        return carry
    lax.fori_loop(0, NPT // 16, zb, 0)
    for k in range(8):
        ones_v[pl.ds(k * 16, 16)] = jnp.ones((16,), jnp.float32)
    pltpu.sync_copy(zv, deg_s.at[pl.ds(s * NPT, NPT)])
    plsc.subcore_barrier()

    pltpu.sync_copy(ei_hbm.at[0, pl.ds(w * CPT, CPT), :], row_v)

    def body(j, carry):
        pltpu.sync_copy(ones_v, deg_s.at[row_v.at[j]], add=True)
        return carry
    lax.fori_loop(0, CPT, body, 0)
    plsc.subcore_barrier()

    @pl.when(s == 0)
    def _():
        pltpu.sync_copy(deg_s, deg_out.at[c])


_deg_kernel = pl.kernel(
    _deg_body,
    out_type=jax.ShapeDtypeStruct((NC, NPAD), jnp.float32),
    mesh=_MESH,
    scratch_types=[
        pltpu.VMEM((CPT, CH), jnp.int32),
        pltpu.VMEM((CH,), jnp.float32),
        pltpu.VMEM((NPT,), jnp.float32),
        pltpu.VMEM_SHARED((NPAD,), jnp.float32),
    ],
)


CPH = 40
NBUF = 2
NGRP = CPH // NBUF
SLOW_CORE = 0
Q_SLOW, Q_FAST = CPH, 3 * CPH


def _prop_body(h_hbm, ei_hbm, out_hbm, row_v, col_v, bufs, acc_s, *sems):
    gsem, ssem = sems[:NBUF], sems[NBUF:]
    c = lax.axis_index("c")
    s = lax.axis_index("s")
    slow = c == SLOW_CORE
    nst = jnp.where(slow, Q_SLOW // CPH, Q_FAST // CPH)
    core_base = jnp.where(slow, s * Q_SLOW, NS * Q_SLOW + s * Q_FAST)

    def zb(r, carry):
        for k in range(8):
            bufs[0, r, pl.ds(k * 16, 16)] = jnp.zeros((16,), jnp.float32)
        return carry
    lax.fori_loop(0, CH, zb, 0)
    for i in range(NPT // CH):
        pltpu.sync_copy(bufs.at[0], acc_s.at[pl.ds(s * NPT + i * CH, CH), :])
    plsc.subcore_barrier()

    def stage(st, carry):
        base = core_base + st * CPH
        pltpu.sync_copy(ei_hbm.at[0, pl.ds(base, CPH), :], row_v)
        pltpu.sync_copy(ei_hbm.at[1, pl.ds(base, CPH), :], col_v)

        for b in range(NBUF):
            pltpu.async_copy(h_hbm.at[row_v.at[b]], bufs.at[b], gsem[b])

        def group(i, carry2):
            for b in range(NBUF):
                j = i * NBUF + b
                pltpu.make_async_copy(h_hbm.at[row_v.at[j]], bufs.at[b],
                                      gsem[b]).wait()
                pltpu.async_copy(bufs.at[b], acc_s.at[col_v.at[j]],
                                 ssem[b], add=True)
            for b in range(NBUF):
                j = i * NBUF + b

                @pl.when(i < NGRP - 1)
                def _():
                    pltpu.make_async_copy(bufs.at[b], acc_s.at[col_v.at[j]],
                                          ssem[b]).wait()
                    pltpu.async_copy(h_hbm.at[row_v.at[j + NBUF]],
                                     bufs.at[b], gsem[b])
            return carry2
        lax.fori_loop(0, NGRP, group, 0)
        for b in range(NBUF):
            j = (NGRP - 1) * NBUF + b
            pltpu.make_async_copy(bufs.at[b], acc_s.at[col_v.at[j]],
                                  ssem[b]).wait()
        return carry
    lax.fori_loop(0, nst, stage, 0)
    plsc.subcore_barrier()

    pltpu.sync_copy(acc_s.at[pl.ds(s * NPT, NPT), :],
                    out_hbm.at[c, pl.ds(s * NPT, NPT), :])


_prop_kernel = pl.kernel(
    _prop_body,
    out_type=jax.ShapeDtypeStruct((NC, NPAD, D), jnp.float32),
    mesh=_MESH,
    scratch_types=[
        pltpu.VMEM((CPH, CH), jnp.int32),
        pltpu.VMEM((CPH, CH), jnp.int32),
        pltpu.VMEM((NBUF, CH, D), jnp.float32),
        pltpu.VMEM_SHARED((NPAD, D), jnp.float32),
    ] + [pltpu.SemaphoreType.DMA] * (2 * NBUF),
)


def _tc1_body(x_ref, degp_ref, w_ref, b_ref, h_ref, dis_ref):
    deg = degp_ref[0, :] + degp_ref[1, :] + 1.0
    dis = lax.rsqrt(deg)
    h = lax.dot_general(x_ref[...], w_ref[...], (((1,), (1,)), ((), ())),
                        preferred_element_type=jnp.float32) + b_ref[...]
    h_ref[...] = h * dis[:, None]
    dis_ref[...] = dis[:, None]


_tc1 = pl.pallas_call(
    _tc1_body,
    grid=(GRID,),
    in_specs=[
        pl.BlockSpec((BLK, D), lambda i: (i, 0)),
        pl.BlockSpec((NC, BLK), lambda i: (0, i)),
        pl.BlockSpec((D, D), lambda i: (0, 0)),
        pl.BlockSpec((1, D), lambda i: (0, 0)),
    ],
    out_specs=[
        pl.BlockSpec((BLK, D), lambda i: (i, 0)),
        pl.BlockSpec((BLK, 1), lambda i: (i, 0)),
    ],
    out_shape=[
        jax.ShapeDtypeStruct((NPAD, D), jnp.float32),
        jax.ShapeDtypeStruct((NPAD, 1), jnp.float32),
    ],
)


def _tc2_body(acc_ref, h1_ref, dis_ref, w_ref, b_ref, o_ref):
    dis = dis_ref[...]
    agg = (acc_ref[0] + acc_ref[1] + h1_ref[...]) * dis
    x2 = jnp.maximum(agg, 0.0)
    h = lax.dot_general(x2, w_ref[...], (((1,), (1,)), ((), ())),
                        preferred_element_type=jnp.float32) + b_ref[...]
    o_ref[...] = h * dis


_tc2 = pl.pallas_call(
    _tc2_body,
    grid=(GRID,),
    in_specs=[
        pl.BlockSpec((NC, BLK, D), lambda i: (0, i, 0)),
        pl.BlockSpec((BLK, D), lambda i: (i, 0)),
        pl.BlockSpec((BLK, 1), lambda i: (i, 0)),
        pl.BlockSpec((D, D), lambda i: (0, 0)),
        pl.BlockSpec((1, D), lambda i: (0, 0)),
    ],
    out_specs=pl.BlockSpec((BLK, D), lambda i: (i, 0)),
    out_shape=jax.ShapeDtypeStruct((NPAD, D), jnp.float32),
)


def _tc3_body(acc_ref, h2_ref, dis_ref, o_ref):
    agg = (acc_ref[0] + acc_ref[1] + h2_ref[...]) * dis_ref[...]
    o_ref[...] = jnp.maximum(agg, 0.0)


_tc3 = pl.pallas_call(
    _tc3_body,
    grid=(GRID,),
    in_specs=[
        pl.BlockSpec((NC, BLK, D), lambda i: (0, i, 0)),
        pl.BlockSpec((BLK, D), lambda i: (i, 0)),
        pl.BlockSpec((BLK, 1), lambda i: (i, 0)),
    ],
    out_specs=pl.BlockSpec((BLK, D), lambda i: (i, 0)),
    out_shape=jax.ShapeDtypeStruct((NPAD, D), jnp.float32),
)


@jax.jit
def kernel(x, edge_index, W1, b1, W2, b2):
    xp = jnp.pad(x[0], ((0, NPAD - N), (0, 0)))
    ei32 = edge_index.astype(jnp.int32)
    pad = jnp.full((2, EPAD - E), N, dtype=jnp.int32)
    eip = jnp.concatenate([ei32, pad], axis=1).reshape(2, NW * CPT, CH)

    deg_parts = _deg_kernel(eip)
    h1p, dis = _tc1(xp, deg_parts, W1, b1.reshape(1, D))
    acc1 = _prop_kernel(h1p, eip)
    h2p = _tc2(acc1, h1p, dis, W2, b2.reshape(1, D))
    acc2 = _prop_kernel(h2p, eip)
    outp = _tc3(acc2, h2p, dis)
    return outp[:N][None]


# speedup vs baseline: 1.0018x; 1.0018x over previous
import functools

import jax
import jax.numpy as jnp
from jax import lax
from jax.experimental import pallas as pl
from jax.experimental.pallas import tpu as pltpu
from jax.experimental.pallas import tpu_sc as plsc

N = 10000
D = 128
E = 320000
NC, NS = 2, 16
NW = NC * NS
CH = 128
CPT = 80
EPAD = NW * CPT * CH
NPAD = 10240
NPT = NPAD // NS
BLK = 640
GRID = NPAD // BLK

_MESH = plsc.VectorSubcoreMesh(
    core_axis_name="c", subcore_axis_name="s", num_cores=NC, num_subcores=NS)


def _deg_body(ei_hbm, deg_out, row_v, ones_v, zv, deg_s):
    c = lax.axis_index("c")
    s = lax.axis_index("s")
    w = c * NS + s

    def zb(r, carry):
        zv[pl.ds(r * 16, 16)] = jnp.zeros((16,), jnp.float32)
        return carry
    lax.fori_loop(0, NPT // 16, zb, 0)
    for k in range(8):
        ones_v[pl.ds(k * 16, 16)] = jnp.ones((16,), jnp.float32)
    pltpu.sync_copy(zv, deg_s.at[pl.ds(s * NPT, NPT)])
    plsc.subcore_barrier()

    pltpu.sync_copy(ei_hbm.at[0, pl.ds(w * CPT, CPT), :], row_v)

    def body(j, carry):
        pltpu.sync_copy(ones_v, deg_s.at[row_v.at[j]], add=True)
        return carry
    lax.fori_loop(0, CPT, body, 0)
    plsc.subcore_barrier()

    @pl.when(s == 0)
    def _():
        pltpu.sync_copy(deg_s, deg_out.at[c])


_deg_kernel = pl.kernel(
    _deg_body,
    out_type=jax.ShapeDtypeStruct((NC, NPAD), jnp.float32),
    mesh=_MESH,
    scratch_types=[
        pltpu.VMEM((CPT, CH), jnp.int32),
        pltpu.VMEM((CH,), jnp.float32),
        pltpu.VMEM((NPT,), jnp.float32),
        pltpu.VMEM_SHARED((NPAD,), jnp.float32),
    ],
)


CPH = 40
NBUF = 2
NGRP = CPH // NBUF
SLOW_CORE = 1
Q_SLOW, Q_FAST = CPH, 3 * CPH


def _prop_body(h_hbm, ei_hbm, out_hbm, row_v, col_v, bufs, acc_s, *sems):
    gsem, ssem = sems[:NBUF], sems[NBUF:]
    c = lax.axis_index("c")
    s = lax.axis_index("s")
    slow = c == SLOW_CORE
    nst = jnp.where(slow, Q_SLOW // CPH, Q_FAST // CPH)
    core_base = jnp.where(slow, s * Q_SLOW, NS * Q_SLOW + s * Q_FAST)

    def zb(r, carry):
        for k in range(8):
            bufs[0, r, pl.ds(k * 16, 16)] = jnp.zeros((16,), jnp.float32)
        return carry
    lax.fori_loop(0, CH, zb, 0)
    for i in range(NPT // CH):
        pltpu.sync_copy(bufs.at[0], acc_s.at[pl.ds(s * NPT + i * CH, CH), :])
    plsc.subcore_barrier()

    def stage(st, carry):
        base = core_base + st * CPH
        pltpu.sync_copy(ei_hbm.at[0, pl.ds(base, CPH), :], row_v)
        pltpu.sync_copy(ei_hbm.at[1, pl.ds(base, CPH), :], col_v)

        for b in range(NBUF):
            pltpu.async_copy(h_hbm.at[row_v.at[b]], bufs.at[b], gsem[b])

        def group(i, carry2):
            for b in range(NBUF):
                j = i * NBUF + b
                pltpu.make_async_copy(h_hbm.at[row_v.at[j]], bufs.at[b],
                                      gsem[b]).wait()
                pltpu.async_copy(bufs.at[b], acc_s.at[col_v.at[j]],
                                 ssem[b], add=True)
            for b in range(NBUF):
                j = i * NBUF + b

                @pl.when(i < NGRP - 1)
                def _():
                    pltpu.make_async_copy(bufs.at[b], acc_s.at[col_v.at[j]],
                                          ssem[b]).wait()
                    pltpu.async_copy(h_hbm.at[row_v.at[j + NBUF]],
                                     bufs.at[b], gsem[b])
            return carry2
        lax.fori_loop(0, NGRP, group, 0)
        for b in range(NBUF):
            j = (NGRP - 1) * NBUF + b
            pltpu.make_async_copy(bufs.at[b], acc_s.at[col_v.at[j]],
                                  ssem[b]).wait()
        return carry
    lax.fori_loop(0, nst, stage, 0)
    plsc.subcore_barrier()

    pltpu.sync_copy(acc_s.at[pl.ds(s * NPT, NPT), :],
                    out_hbm.at[c, pl.ds(s * NPT, NPT), :])


_prop_kernel = pl.kernel(
    _prop_body,
    out_type=jax.ShapeDtypeStruct((NC, NPAD, D), jnp.float32),
    mesh=_MESH,
    scratch_types=[
        pltpu.VMEM((CPH, CH), jnp.int32),
        pltpu.VMEM((CPH, CH), jnp.int32),
        pltpu.VMEM((NBUF, CH, D), jnp.float32),
        pltpu.VMEM_SHARED((NPAD, D), jnp.float32),
    ] + [pltpu.SemaphoreType.DMA] * (2 * NBUF),
)


def _tc1_body(x_ref, degp_ref, w_ref, b_ref, h_ref, dis_ref):
    deg = degp_ref[0, :] + degp_ref[1, :] + 1.0
    dis = lax.rsqrt(deg)
    h = lax.dot_general(x_ref[...], w_ref[...], (((1,), (1,)), ((), ())),
                        preferred_element_type=jnp.float32) + b_ref[...]
    h_ref[...] = h * dis[:, None]
    dis_ref[...] = dis[:, None]


_tc1 = pl.pallas_call(
    _tc1_body,
    grid=(GRID,),
    in_specs=[
        pl.BlockSpec((BLK, D), lambda i: (i, 0)),
        pl.BlockSpec((NC, BLK), lambda i: (0, i)),
        pl.BlockSpec((D, D), lambda i: (0, 0)),
        pl.BlockSpec((1, D), lambda i: (0, 0)),
    ],
    out_specs=[
        pl.BlockSpec((BLK, D), lambda i: (i, 0)),
        pl.BlockSpec((BLK, 1), lambda i: (i, 0)),
    ],
    out_shape=[
        jax.ShapeDtypeStruct((NPAD, D), jnp.float32),
        jax.ShapeDtypeStruct((NPAD, 1), jnp.float32),
    ],
)


def _tc2_body(acc_ref, h1_ref, dis_ref, w_ref, b_ref, o_ref):
    dis = dis_ref[...]
    agg = (acc_ref[0] + acc_ref[1] + h1_ref[...]) * dis
    x2 = jnp.maximum(agg, 0.0)
    h = lax.dot_general(x2, w_ref[...], (((1,), (1,)), ((), ())),
                        preferred_element_type=jnp.float32) + b_ref[...]
    o_ref[...] = h * dis


_tc2 = pl.pallas_call(
    _tc2_body,
    grid=(GRID,),
    in_specs=[
        pl.BlockSpec((NC, BLK, D), lambda i: (0, i, 0)),
        pl.BlockSpec((BLK, D), lambda i: (i, 0)),
        pl.BlockSpec((BLK, 1), lambda i: (i, 0)),
        pl.BlockSpec((D, D), lambda i: (0, 0)),
        pl.BlockSpec((1, D), lambda i: (0, 0)),
    ],
    out_specs=pl.BlockSpec((BLK, D), lambda i: (i, 0)),
    out_shape=jax.ShapeDtypeStruct((NPAD, D), jnp.float32),
)


def _tc3_body(acc_ref, h2_ref, dis_ref, o_ref):
    agg = (acc_ref[0] + acc_ref[1] + h2_ref[...]) * dis_ref[...]
    o_ref[...] = jnp.maximum(agg, 0.0)


_tc3 = pl.pallas_call(
    _tc3_body,
    grid=(GRID,),
    in_specs=[
        pl.BlockSpec((NC, BLK, D), lambda i: (0, i, 0)),
        pl.BlockSpec((BLK, D), lambda i: (i, 0)),
        pl.BlockSpec((BLK, 1), lambda i: (i, 0)),
    ],
    out_specs=pl.BlockSpec((BLK, D), lambda i: (i, 0)),
    out_shape=jax.ShapeDtypeStruct((NPAD, D), jnp.float32),
)


@jax.jit
def kernel(x, edge_index, W1, b1, W2, b2):
    xp = jnp.pad(x[0], ((0, NPAD - N), (0, 0)))
    ei32 = edge_index.astype(jnp.int32)
    pad = jnp.full((2, EPAD - E), N, dtype=jnp.int32)
    eip = jnp.concatenate([ei32, pad], axis=1).reshape(2, NW * CPT, CH)

    deg_parts = _deg_kernel(eip)
    h1p, dis = _tc1(xp, deg_parts, W1, b1.reshape(1, D))
    acc1 = _prop_kernel(h1p, eip)
    h2p = _tc2(acc1, h1p, dis, W2, b2.reshape(1, D))
    acc2 = _prop_kernel(h2p, eip)
    outp = _tc3(acc2, h2p, dis)
    return outp[:N][None]


# speedup vs baseline: 1.5680x; 1.5652x over previous
import functools

import jax
import jax.numpy as jnp
from jax import lax
from jax.experimental import pallas as pl
from jax.experimental.pallas import tpu as pltpu
from jax.experimental.pallas import tpu_sc as plsc

N = 10000
D = 128
E = 320000
NC, NS = 2, 16
NW = NC * NS
CH = 128
CPT = 80
EPAD = NW * CPT * CH
NPAD = 10240
NPT = NPAD // NS
BLK = 640
GRID = NPAD // BLK

_MESH = plsc.VectorSubcoreMesh(
    core_axis_name="c", subcore_axis_name="s", num_cores=NC, num_subcores=NS)


def _deg_body(ei_hbm, deg_out, row_v, ones_v, zv, deg_s):
    c = lax.axis_index("c")
    s = lax.axis_index("s")
    w = c * NS + s

    def zb(r, carry):
        zv[pl.ds(r * 16, 16)] = jnp.zeros((16,), jnp.float32)
        return carry
    lax.fori_loop(0, NPT // 16, zb, 0)
    for k in range(8):
        ones_v[pl.ds(k * 16, 16)] = jnp.ones((16,), jnp.float32)
    pltpu.sync_copy(zv, deg_s.at[pl.ds(s * NPT, NPT)])
    plsc.subcore_barrier()

    pltpu.sync_copy(ei_hbm.at[0, pl.ds(w * CPT, CPT), :], row_v)

    def body(j, carry):
        pltpu.sync_copy(ones_v, deg_s.at[row_v.at[j]], add=True)
        return carry
    lax.fori_loop(0, CPT, body, 0)
    plsc.subcore_barrier()

    @pl.when(s == 0)
    def _():
        pltpu.sync_copy(deg_s, deg_out.at[c])


_deg_kernel = pl.kernel(
    _deg_body,
    out_type=jax.ShapeDtypeStruct((NC, NPAD), jnp.float32),
    mesh=_MESH,
    scratch_types=[
        pltpu.VMEM((CPT, CH), jnp.int32),
        pltpu.VMEM((CH,), jnp.float32),
        pltpu.VMEM((NPT,), jnp.float32),
        pltpu.VMEM_SHARED((NPAD,), jnp.float32),
    ],
)


PCH = 64
PCPT = EPAD // (NW * PCH)
SCH = 16
NST = PCPT // SCH
NBUF = 2
NGRP = SCH // NBUF
HW = D // 2


def _prop_body(h_hbm, ei_hbm, out_hbm, row_v, col_v, gbuf, sbuf, acc_s,
               *sems):
    gsem, ssem = sems[:NBUF], sems[NBUF:]
    c = lax.axis_index("c")
    s = lax.axis_index("s")
    w = c * NS + s

    def zb(r, carry):
        for k in range(8):
            sbuf[0, r, pl.ds(k * 16, 16)] = jnp.zeros((16,), jnp.float32)
        return carry
    lax.fori_loop(0, PCH, zb, 0)
    for i in range(NPT // PCH):
        pltpu.sync_copy(sbuf.at[0], acc_s.at[pl.ds(s * NPT + i * PCH, PCH), :])
    plsc.subcore_barrier()

    def unpack_chunk(b):
        himask = jnp.full((16,), -65536, jnp.int32)

        def rowfn(r, carry):
            for k in range(4):
                wv = gbuf[b, r, pl.ds(k * 16, 16)]
                lo = lax.bitcast_convert_type(
                    lax.shift_left(wv, 16), jnp.float32)
                hi = lax.bitcast_convert_type(
                    lax.bitwise_and(wv, himask), jnp.float32)
                sbuf[b, r, pl.ds(k * 32, 16)] = lo
                sbuf[b, r, pl.ds(k * 32 + 16, 16)] = hi
            return carry
        lax.fori_loop(0, PCH, rowfn, 0)

    def stage(st, carry):
        base = w * PCPT + st * SCH
        pltpu.sync_copy(ei_hbm.at[0, pl.ds(base, SCH), :], row_v)
        pltpu.sync_copy(ei_hbm.at[1, pl.ds(base, SCH), :], col_v)

        for b in range(NBUF):
            pltpu.async_copy(h_hbm.at[row_v.at[b]], gbuf.at[b], gsem[b])

        def group(i, carry2):
            for b in range(NBUF):
                j = i * NBUF + b
                pltpu.make_async_copy(h_hbm.at[row_v.at[j]], gbuf.at[b],
                                      gsem[b]).wait()

                @pl.when(i > 0)
                def _():
                    pltpu.make_async_copy(sbuf.at[b], acc_s.at[col_v.at[j]],
                                          ssem[b]).wait()
                unpack_chunk(b)

                @pl.when(i < NGRP - 1)
                def _():
                    pltpu.async_copy(h_hbm.at[row_v.at[j + NBUF]], gbuf.at[b],
                                     gsem[b])
                pltpu.async_copy(sbuf.at[b], acc_s.at[col_v.at[j]], ssem[b],
                                 add=True)
            return carry2
        lax.fori_loop(0, NGRP, group, 0)
        for b in range(NBUF):
            j = (NGRP - 1) * NBUF + b
            pltpu.make_async_copy(sbuf.at[b], acc_s.at[col_v.at[j]],
                                  ssem[b]).wait()
        return carry
    lax.fori_loop(0, NST, stage, 0)
    plsc.subcore_barrier()

    pltpu.sync_copy(acc_s.at[pl.ds(s * NPT, NPT), :],
                    out_hbm.at[c, pl.ds(s * NPT, NPT), :])


_prop_kernel = pl.kernel(
    _prop_body,
    out_type=jax.ShapeDtypeStruct((NC, NPAD, D), jnp.float32),
    mesh=_MESH,
    compiler_params=pltpu.CompilerParams(use_tc_tiling_on_sc=False),
    scratch_types=[
        pltpu.VMEM((SCH, PCH), jnp.int32),
        pltpu.VMEM((SCH, PCH), jnp.int32),
        pltpu.VMEM((NBUF, PCH, HW), jnp.int32),
        pltpu.VMEM((NBUF, PCH, D), jnp.float32),
        pltpu.VMEM_SHARED((NPAD, D), jnp.float32),
    ] + [pltpu.SemaphoreType.DMA] * (2 * NBUF),
)


def _tc1_body(x_ref, degp_ref, w_ref, b_ref, h_ref, dis_ref):
    deg = degp_ref[0, :] + degp_ref[1, :] + 1.0
    dis = lax.rsqrt(deg)
    h = lax.dot_general(x_ref[...], w_ref[...], (((1,), (1,)), ((), ())),
                        preferred_element_type=jnp.float32) + b_ref[...]
    h_ref[...] = h * dis[:, None]
    dis_ref[...] = dis[:, None]


_tc1 = pl.pallas_call(
    _tc1_body,
    grid=(GRID,),
    in_specs=[
        pl.BlockSpec((BLK, D), lambda i: (i, 0)),
        pl.BlockSpec((NC, BLK), lambda i: (0, i)),
        pl.BlockSpec((D, D), lambda i: (0, 0)),
        pl.BlockSpec((1, D), lambda i: (0, 0)),
    ],
    out_specs=[
        pl.BlockSpec((BLK, D), lambda i: (i, 0)),
        pl.BlockSpec((BLK, 1), lambda i: (i, 0)),
    ],
    out_shape=[
        jax.ShapeDtypeStruct((NPAD, D), jnp.float32),
        jax.ShapeDtypeStruct((NPAD, 1), jnp.float32),
    ],
)


def _tc2_body(acc_ref, h1_ref, dis_ref, w_ref, b_ref, o_ref):
    dis = dis_ref[...]
    agg = (acc_ref[0] + acc_ref[1] + h1_ref[...]) * dis
    x2 = jnp.maximum(agg, 0.0)
    h = lax.dot_general(x2, w_ref[...], (((1,), (1,)), ((), ())),
                        preferred_element_type=jnp.float32) + b_ref[...]
    o_ref[...] = h * dis


_tc2 = pl.pallas_call(
    _tc2_body,
    grid=(GRID,),
    in_specs=[
        pl.BlockSpec((NC, BLK, D), lambda i: (0, i, 0)),
        pl.BlockSpec((BLK, D), lambda i: (i, 0)),
        pl.BlockSpec((BLK, 1), lambda i: (i, 0)),
        pl.BlockSpec((D, D), lambda i: (0, 0)),
        pl.BlockSpec((1, D), lambda i: (0, 0)),
    ],
    out_specs=pl.BlockSpec((BLK, D), lambda i: (i, 0)),
    out_shape=jax.ShapeDtypeStruct((NPAD, D), jnp.float32),
)


def _tc3_body(acc_ref, h2_ref, dis_ref, o_ref):
    agg = (acc_ref[0] + acc_ref[1] + h2_ref[...]) * dis_ref[...]
    o_ref[...] = jnp.maximum(agg, 0.0)


_tc3 = pl.pallas_call(
    _tc3_body,
    grid=(GRID,),
    in_specs=[
        pl.BlockSpec((NC, BLK, D), lambda i: (0, i, 0)),
        pl.BlockSpec((BLK, D), lambda i: (i, 0)),
        pl.BlockSpec((BLK, 1), lambda i: (i, 0)),
    ],
    out_specs=pl.BlockSpec((BLK, D), lambda i: (i, 0)),
    out_shape=jax.ShapeDtypeStruct((NPAD, D), jnp.float32),
)


_PERM = jnp.asarray(
    [32 * g + (k // 2) + 16 * (k % 2) for g in range(4) for k in range(32)],
    dtype=jnp.int32)


def _pack_table(h):
    hp = jnp.take(h, _PERM, axis=1).astype(jnp.bfloat16)
    return jax.lax.bitcast_convert_type(hp.reshape(NPAD, HW, 2), jnp.int32)


@jax.jit
def kernel(x, edge_index, W1, b1, W2, b2):
    xp = jnp.pad(x[0], ((0, NPAD - N), (0, 0)))
    ei32 = edge_index.astype(jnp.int32)
    pad = jnp.full((2, EPAD - E), N, dtype=jnp.int32)
    eif = jnp.concatenate([ei32, pad], axis=1)
    eip = eif.reshape(2, NW * CPT, CH)
    eip64 = eif.reshape(2, NW * PCPT, PCH)

    deg_parts = _deg_kernel(eip)
    h1p, dis = _tc1(xp, deg_parts, W1, b1.reshape(1, D))
    acc1 = _prop_kernel(_pack_table(h1p), eip64)
    h2p = _tc2(acc1, h1p, dis, W2, b2.reshape(1, D))
    acc2 = _prop_kernel(_pack_table(h2p), eip64)
    outp = _tc3(acc2, h2p, dis)
    return outp[:N][None]
